# Initial kernel scaffold; baseline (speedup 1.0000x reference)
import jax
import jax.numpy as jnp
from jax.experimental import pallas as pl


def kernel(rtype_idx, arg_dt_idx, arg_const_idx, stmt_dt_idx, const_idx, func_class_idx, func_func_idx, dt_table, const_table, class_table, func_table):
    raise NotImplementedError("write your pallas kernel here")


# speedup vs baseline: 3.1537x; 3.1537x over previous
import functools

import jax
import jax.numpy as jnp
from jax import lax
from jax.experimental import pallas as pl
from jax.experimental.pallas import tpu as pltpu
from jax.experimental.pallas import tpu_sc as plsc

B = 16384
A = 8
D = 64
CLASS_D = 16
FUNC_D = 48
MAX_NORM = 2.0

NC = 2
NS = 16
NW = NC * NS
BP = B // NW
C = 16
NCHUNK = BP // C

W_RT = 0.5
W_AD = 0.75 / 16.0
W_ST = 1.0 / 16.0
W_AC = 0.25 / 16.0
W_CI = 1.0 / 16.0
W_CF = 1.0 / 16.0


def _renorm_body(seg, x_ref, o_ref):
    e = x_ref[...]
    w = e.shape[-1]
    r = lax.broadcasted_iota(jnp.int32, (w, w), 0) // seg
    c = lax.broadcasted_iota(jnp.int32, (w, w), 1) // seg
    m = (r == c).astype(jnp.float32)
    s = lax.dot(e * e, m, precision=lax.Precision.HIGHEST)
    n = jnp.sqrt(s)
    scale = jnp.where(n > MAX_NORM, MAX_NORM / (n + 1e-7), 1.0)
    o_ref[...] = e * scale


def _renorm_table(t, width, block_rows):
    seg = t.shape[-1]
    rows = t.size // width
    t2 = t.reshape(rows, width)
    grid = pl.cdiv(rows, block_rows)
    out = pl.pallas_call(
        functools.partial(_renorm_body, seg),
        grid=(grid,),
        in_specs=[pl.BlockSpec((block_rows, width), lambda i: (i, 0))],
        out_specs=pl.BlockSpec((block_rows, width), lambda i: (i, 0)),
        out_shape=jax.ShapeDtypeStruct((rows, width), jnp.float32),
    )(t2)
    return out.reshape(t.shape)


def _sc_body(rt_hbm, ad_hbm, ac_hbm, sd_hbm, ci_hbm, fc_hbm, ff_hbm,
             dtn_hbm, cn_hbm, cln_hbm, fnn_hbm, out_hbm,
             rt_idx, ad_idx, ac_idx, sd_idx, ci_idx, fc_idx, ff_idx,
             rt_rows, ad_rows, ac_rows, sd_rows, ci_rows, fc_rows, ff_rows,
             out_buf, sem):
    wid = lax.axis_index("s") * NC + lax.axis_index("c")

    def chunk(g, carry):
        base = wid * BP + g * C
        base8 = base * A
        pltpu.sync_copy(rt_hbm.at[pl.ds(base, C)], rt_idx)
        pltpu.sync_copy(ad_hbm.at[pl.ds(base8, C * A)], ad_idx)
        pltpu.sync_copy(ac_hbm.at[pl.ds(base8, C * A)], ac_idx)
        pltpu.sync_copy(sd_hbm.at[pl.ds(base8, C * A)], sd_idx)
        pltpu.sync_copy(ci_hbm.at[pl.ds(base8, C * A)], ci_idx)
        pltpu.sync_copy(fc_hbm.at[pl.ds(base8, C * A)], fc_idx)
        pltpu.sync_copy(ff_hbm.at[pl.ds(base8, C * A)], ff_idx)

        cps = [
            pltpu.async_copy(dtn_hbm.at[rt_idx], rt_rows, sem),
            pltpu.async_copy(dtn_hbm.at[ad_idx], ad_rows, sem),
            pltpu.async_copy(cn_hbm.at[ac_idx], ac_rows, sem),
            pltpu.async_copy(dtn_hbm.at[sd_idx], sd_rows, sem),
            pltpu.async_copy(cn_hbm.at[ci_idx], ci_rows, sem),
            pltpu.async_copy(cln_hbm.at[fc_idx], fc_rows, sem),
            pltpu.async_copy(fnn_hbm.at[ff_idx], ff_rows, sem),
        ]
        for cp in cps:
            cp.wait()

        def row(i, carry2):
            for j in range(4):
                js = pl.ds(16 * j, 16)
                acc0 = rt_rows[i, js] * W_RT
                acc1 = jnp.zeros((16,), jnp.float32)
                for a in range(A):
                    k = i * A + a
                    if j == 0:
                        t = fc_rows[k, :] * W_CF
                    else:
                        t = ff_rows[k, pl.ds(16 * (j - 1), 16)] * W_CF
                    t = t + ad_rows[k, js] * W_AD
                    t = t + sd_rows[k, js] * W_ST
                    u = ac_rows[k, js] * W_AC
                    u = u + ci_rows[k, js] * W_CI
                    if a % 2 == 0:
                        acc0 = acc0 + (t + u)
                    else:
                        acc1 = acc1 + (t + u)
                out_buf[i, js] = acc0 + acc1
            return carry2

        lax.fori_loop(0, C, row, 0, unroll=False)
        pltpu.sync_copy(out_buf, out_hbm.at[pl.ds(base, C)])
        return carry

    lax.fori_loop(0, NCHUNK, chunk, 0, unroll=False)


def _sc_embed(rt, ad, ac, sd, ci, fc, ff, dtn, cn, cln, fnn):
    mesh = plsc.VectorSubcoreMesh(
        core_axis_name="c", subcore_axis_name="s",
        num_cores=NC, num_subcores=NS)
    f = pl.kernel(
        _sc_body,
        out_type=jax.ShapeDtypeStruct((B, D), jnp.float32),
        mesh=mesh,
        scratch_types=[
            pltpu.VMEM((C,), jnp.int32),
            pltpu.VMEM((C * A,), jnp.int32),
            pltpu.VMEM((C * A,), jnp.int32),
            pltpu.VMEM((C * A,), jnp.int32),
            pltpu.VMEM((C * A,), jnp.int32),
            pltpu.VMEM((C * A,), jnp.int32),
            pltpu.VMEM((C * A,), jnp.int32),
            pltpu.VMEM((C, D), jnp.float32),
            pltpu.VMEM((C * A, D), jnp.float32),
            pltpu.VMEM((C * A, D), jnp.float32),
            pltpu.VMEM((C * A, D), jnp.float32),
            pltpu.VMEM((C * A, D), jnp.float32),
            pltpu.VMEM((C * A, CLASS_D), jnp.float32),
            pltpu.VMEM((C * A, FUNC_D), jnp.float32),
            pltpu.VMEM((C, D), jnp.float32),
            pltpu.SemaphoreType.DMA,
        ],
        compiler_params=pltpu.CompilerParams(use_tc_tiling_on_sc=False),
    )
    return f(rt, ad, ac, sd, ci, fc, ff, dtn, cn, cln, fnn)


def kernel(rtype_idx, arg_dt_idx, arg_const_idx, stmt_dt_idx, const_idx,
           func_class_idx, func_func_idx, dt_table, const_table,
           class_table, func_table):
    dtn = _renorm_table(dt_table, 128, 512)
    cn = _renorm_table(const_table, 128, 2048)
    cln = _renorm_table(class_table, 128, 2048)
    fnn = _renorm_table(func_table, 384, 2048)

    i32 = jnp.int32
    rt = rtype_idx.astype(i32)
    ad = arg_dt_idx.reshape(-1).astype(i32)
    ac = arg_const_idx.reshape(-1).astype(i32)
    sd = stmt_dt_idx.reshape(-1).astype(i32)
    ci = const_idx.reshape(-1).astype(i32)
    fc = func_class_idx.reshape(-1).astype(i32)
    ff = func_func_idx.reshape(-1).astype(i32)

    return _sc_embed(rt, ad, ac, sd, ci, fc, ff, dtn, cn, cln, fnn)


# speedup vs baseline: 3.6360x; 1.1529x over previous
import functools

import jax
import jax.numpy as jnp
from jax import lax
from jax.experimental import pallas as pl
from jax.experimental.pallas import tpu as pltpu
from jax.experimental.pallas import tpu_sc as plsc

B = 16384
A = 8
D = 64
CLASS_D = 16
FUNC_D = 48
MAX_NORM = 2.0

NC = 2
NS = 16
NW = NC * NS
BP = B // NW
C = 16
CA = C * A
NCHUNK = BP // C

W_RT = 0.5
W_AD = 0.75 / 16.0
W_ST = 1.0 / 16.0
W_AC = 0.25 / 16.0
W_CI = 1.0 / 16.0
W_CF = 1.0 / 16.0


def _renorm_body(seg, x_ref, o_ref):
    e = x_ref[...]
    w = e.shape[-1]
    r = lax.broadcasted_iota(jnp.int32, (w, w), 0) // seg
    c = lax.broadcasted_iota(jnp.int32, (w, w), 1) // seg
    m = (r == c).astype(jnp.float32)
    s = lax.dot(e * e, m, precision=lax.Precision.HIGHEST)
    n = jnp.sqrt(s)
    scale = jnp.where(n > MAX_NORM, MAX_NORM / (n + 1e-7), 1.0)
    o_ref[...] = e * scale


def _renorm_table(t, width, block_rows):
    seg = t.shape[-1]
    rows = t.size // width
    t2 = t.reshape(rows, width)
    grid = pl.cdiv(rows, block_rows)
    out = pl.pallas_call(
        functools.partial(_renorm_body, seg),
        grid=(grid,),
        in_specs=[pl.BlockSpec((block_rows, width), lambda i: (i, 0))],
        out_specs=pl.BlockSpec((block_rows, width), lambda i: (i, 0)),
        out_shape=jax.ShapeDtypeStruct((rows, width), jnp.float32),
    )(t2)
    return out.reshape(t.shape)


def _sc_body(rt_hbm, ad_hbm, ac_hbm, sd_hbm, ci_hbm, fc_hbm, ff_hbm,
             dtn_hbm, cn_hbm, cln_hbm, fnn_hbm, out_hbm,
             rt_ix, ad_ix, ac_ix, sd_ix, ci_ix, fc_ix, ff_ix,
             rt_r0, ad_r0, ac_r0, sd_r0, ci_r0, fc_r0, ff_r0,
             rt_r1, ad_r1, ac_r1, sd_r1, ci_r1, fc_r1, ff_r1,
             ob0, ob1, gsem0, gsem1, osem0, osem1):
    wid = lax.axis_index("s") * NC + lax.axis_index("c")

    pltpu.sync_copy(rt_hbm.at[wid], rt_ix)
    pltpu.sync_copy(ad_hbm.at[wid], ad_ix)
    pltpu.sync_copy(ac_hbm.at[wid], ac_ix)
    pltpu.sync_copy(sd_hbm.at[wid], sd_ix)
    pltpu.sync_copy(ci_hbm.at[wid], ci_ix)
    pltpu.sync_copy(fc_hbm.at[wid], fc_ix)
    pltpu.sync_copy(ff_hbm.at[wid], ff_ix)

    bufs = ((rt_r0, ad_r0, ac_r0, sd_r0, ci_r0, fc_r0, ff_r0),
            (rt_r1, ad_r1, ac_r1, sd_r1, ci_r1, fc_r1, ff_r1))
    obufs = (ob0, ob1)
    gsems = (gsem0, gsem1)
    osems = (osem0, osem1)

    def gathers(g, slot):
        rt_r, ad_r, ac_r, sd_r, ci_r, fc_r, ff_r = bufs[slot]
        return (
            (dtn_hbm.at[rt_ix.at[g]], rt_r),
            (dtn_hbm.at[ad_ix.at[g]], ad_r),
            (cn_hbm.at[ac_ix.at[g]], ac_r),
            (dtn_hbm.at[sd_ix.at[g]], sd_r),
            (cn_hbm.at[ci_ix.at[g]], ci_r),
            (cln_hbm.at[fc_ix.at[g]], fc_r),
            (fnn_hbm.at[ff_ix.at[g]], ff_r),
        )

    def issue(g, slot):
        for s, d in gathers(g, slot):
            pltpu.async_copy(s, d, gsems[slot])

    def drain(g, slot):
        for s, d in gathers(g, slot):
            pltpu.make_async_copy(s, d, gsems[slot]).wait()

    def accumulate(slot):
        rt_r, ad_r, ac_r, sd_r, ci_r, fc_r, ff_r = bufs[slot]
        ob = obufs[slot]

        def row(i, c2):
            for j in range(4):
                js = pl.ds(16 * j, 16)
                acc0 = rt_r[i, js] * W_RT
                acc1 = jnp.zeros((16,), jnp.float32)
                for a in range(A):
                    k = i * A + a
                    if j == 0:
                        t = fc_r[k, :] * W_CF
                    else:
                        t = ff_r[k, pl.ds(16 * (j - 1), 16)] * W_CF
                    t = t + ad_r[k, js] * W_AD
                    t = t + sd_r[k, js] * W_ST
                    u = ac_r[k, js] * W_AC
                    u = u + ci_r[k, js] * W_CI
                    if a % 2 == 0:
                        acc0 = acc0 + (t + u)
                    else:
                        acc1 = acc1 + (t + u)
                ob[i, js] = acc0 + acc1
            return c2

        lax.fori_loop(0, C, row, 0, unroll=False)

    def out_slice(g):
        return out_hbm.at[pl.ds(wid * BP + g * C, C)]

    def body(g, carry):
        cps = [pltpu.async_copy(s, d, gsem0) for s, d in gathers(g, 0)]
        for cp in cps:
            cp.wait()
        accumulate(0)
        pltpu.sync_copy(ob0, out_slice(g))
        return carry

    lax.fori_loop(0, NCHUNK, body, 0, unroll=False)


def _sc_embed(rt, ad, ac, sd, ci, fc, ff, dtn, cn, cln, fnn):
    mesh = plsc.VectorSubcoreMesh(
        core_axis_name="c", subcore_axis_name="s",
        num_cores=NC, num_subcores=NS)
    row_bufs = [
        pltpu.VMEM((C, D), jnp.float32),
        pltpu.VMEM((CA, D), jnp.float32),
        pltpu.VMEM((CA, D), jnp.float32),
        pltpu.VMEM((CA, D), jnp.float32),
        pltpu.VMEM((CA, D), jnp.float32),
        pltpu.VMEM((CA, CLASS_D), jnp.float32),
        pltpu.VMEM((CA, FUNC_D), jnp.float32),
    ]
    f = pl.kernel(
        _sc_body,
        out_type=jax.ShapeDtypeStruct((B, D), jnp.float32),
        mesh=mesh,
        scratch_types=[
            pltpu.VMEM((NCHUNK, C), jnp.int32),
            pltpu.VMEM((NCHUNK, CA), jnp.int32),
            pltpu.VMEM((NCHUNK, CA), jnp.int32),
            pltpu.VMEM((NCHUNK, CA), jnp.int32),
            pltpu.VMEM((NCHUNK, CA), jnp.int32),
            pltpu.VMEM((NCHUNK, CA), jnp.int32),
            pltpu.VMEM((NCHUNK, CA), jnp.int32),
            *row_bufs,
            *row_bufs,
            pltpu.VMEM((C, D), jnp.float32),
            pltpu.VMEM((C, D), jnp.float32),
            pltpu.SemaphoreType.DMA,
            pltpu.SemaphoreType.DMA,
            pltpu.SemaphoreType.DMA,
            pltpu.SemaphoreType.DMA,
        ],
        compiler_params=pltpu.CompilerParams(use_tc_tiling_on_sc=False),
    )
    return f(rt, ad, ac, sd, ci, fc, ff, dtn, cn, cln, fnn)


def kernel(rtype_idx, arg_dt_idx, arg_const_idx, stmt_dt_idx, const_idx,
           func_class_idx, func_func_idx, dt_table, const_table,
           class_table, func_table):
    dtn = _renorm_table(dt_table, 128, 512)
    cn = _renorm_table(const_table, 128, 2048)
    cln = _renorm_table(class_table, 128, 2048)
    fnn = _renorm_table(func_table, 384, 2048)

    i32 = jnp.int32
    rt = rtype_idx.astype(i32).reshape(NW, NCHUNK, C)
    ad = arg_dt_idx.astype(i32).reshape(NW, NCHUNK, CA)
    ac = arg_const_idx.astype(i32).reshape(NW, NCHUNK, CA)
    sd = stmt_dt_idx.astype(i32).reshape(NW, NCHUNK, CA)
    ci = const_idx.astype(i32).reshape(NW, NCHUNK, CA)
    fc = func_class_idx.astype(i32).reshape(NW, NCHUNK, CA)
    ff = func_func_idx.astype(i32).reshape(NW, NCHUNK, CA)

    return _sc_embed(rt, ad, ac, sd, ci, fc, ff, dtn, cn, cln, fnn)


# speedup vs baseline: 4.2015x; 1.1555x over previous
import functools

import jax
import jax.numpy as jnp
from jax import lax
from jax.experimental import pallas as pl
from jax.experimental.pallas import tpu as pltpu
from jax.experimental.pallas import tpu_sc as plsc

B = 16384
A = 8
D = 64
CLASS_D = 16
FUNC_D = 48
MAX_NORM = 2.0

NC = 2
NS = 16
NW = NC * NS
BP = B // NW
C = 16
CA = C * A
NCHUNK = BP // C

W_RT = 0.5
W_AD = 0.75 / 16.0
W_ST = 1.0 / 16.0
W_AC = 0.25 / 16.0
W_CI = 1.0 / 16.0
W_CF = 1.0 / 16.0


def _renorm_body(seg, x_ref, o_ref):
    e = x_ref[...]
    w = e.shape[-1]
    r = lax.broadcasted_iota(jnp.int32, (w, w), 0) // seg
    c = lax.broadcasted_iota(jnp.int32, (w, w), 1) // seg
    m = (r == c).astype(jnp.float32)
    s = lax.dot(e * e, m, precision=lax.Precision.HIGHEST)
    n = jnp.sqrt(s)
    scale = jnp.where(n > MAX_NORM, MAX_NORM / (n + 1e-7), 1.0)
    o_ref[...] = e * scale


def _renorm_table(t, width, block_rows):
    seg = t.shape[-1]
    rows = t.size // width
    t2 = t.reshape(rows, width)
    grid = pl.cdiv(rows, block_rows)
    out = pl.pallas_call(
        functools.partial(_renorm_body, seg),
        grid=(grid,),
        in_specs=[pl.BlockSpec((block_rows, width), lambda i: (i, 0))],
        out_specs=pl.BlockSpec((block_rows, width), lambda i: (i, 0)),
        out_shape=jax.ShapeDtypeStruct((rows, width), jnp.float32),
    )(t2)
    return out.reshape(t.shape)


def _sc_body(rt_hbm, ad_hbm, ac_hbm, sd_hbm, ci_hbm, fc_hbm, ff_hbm,
             dtn_hbm, cn_hbm, cln_hbm, fnn_hbm, out_hbm,
             rt_ix, ad_ix, ac_ix, sd_ix, ci_ix, fc_ix, ff_ix,
             rt_r0, ad_r0, ac_r0, sd_r0, ci_r0, fc_r0, ff_r0,
             rt_r1, ad_r1, ac_r1, sd_r1, ci_r1, fc_r1, ff_r1,
             ob0, ob1, gsem0, gsem1, osem0, osem1):
    wid = lax.axis_index("s") * NC + lax.axis_index("c")

    pltpu.sync_copy(rt_hbm.at[wid], rt_ix)
    pltpu.sync_copy(ad_hbm.at[wid], ad_ix)
    pltpu.sync_copy(ac_hbm.at[wid], ac_ix)
    pltpu.sync_copy(sd_hbm.at[wid], sd_ix)
    pltpu.sync_copy(ci_hbm.at[wid], ci_ix)
    pltpu.sync_copy(fc_hbm.at[wid], fc_ix)
    pltpu.sync_copy(ff_hbm.at[wid], ff_ix)

    bufs = ((rt_r0, ad_r0, ac_r0, sd_r0, ci_r0, fc_r0, ff_r0),
            (rt_r1, ad_r1, ac_r1, sd_r1, ci_r1, fc_r1, ff_r1))
    obufs = (ob0, ob1)
    gsems = (gsem0, gsem1)
    osems = (osem0, osem1)

    def gathers(g, slot):
        rt_r, ad_r, ac_r, sd_r, ci_r, fc_r, ff_r = bufs[slot]
        return (
            (dtn_hbm.at[rt_ix.at[g]], rt_r),
            (dtn_hbm.at[ad_ix.at[g]], ad_r),
            (cn_hbm.at[ac_ix.at[g]], ac_r),
            (dtn_hbm.at[sd_ix.at[g]], sd_r),
            (cn_hbm.at[ci_ix.at[g]], ci_r),
            (cln_hbm.at[fc_ix.at[g]], fc_r),
            (fnn_hbm.at[ff_ix.at[g]], ff_r),
        )

    def issue(g, slot):
        for s, d in gathers(g, slot):
            pltpu.async_copy(s, d, gsems[slot])

    def drain(g, slot):
        for s, d in gathers(g, slot):
            pltpu.make_async_copy(s, d, gsems[slot]).wait()

    def accumulate(slot):
        rt_r, ad_r, ac_r, sd_r, ci_r, fc_r, ff_r = bufs[slot]
        ob = obufs[slot]

        def row(i, c2):
            for j in range(4):
                js = pl.ds(16 * j, 16)
                acc0 = rt_r[i, js] * W_RT
                acc1 = jnp.zeros((16,), jnp.float32)
                for a in range(A):
                    k = i * A + a
                    if j == 0:
                        t = fc_r[k, :] * W_CF
                    else:
                        t = ff_r[k, pl.ds(16 * (j - 1), 16)] * W_CF
                    t = t + ad_r[k, js] * W_AD
                    t = t + sd_r[k, js] * W_ST
                    u = ac_r[k, js] * W_AC
                    u = u + ci_r[k, js] * W_CI
                    if a % 2 == 0:
                        acc0 = acc0 + (t + u)
                    else:
                        acc1 = acc1 + (t + u)
                ob[i, js] = acc0 + acc1
            return c2

        lax.fori_loop(0, C, row, 0, unroll=False)

    def out_slice(g):
        return out_hbm.at[pl.ds(wid * BP + g * C, C)]

    def half(g, slot):
        drain(g, slot)
        accumulate(slot)
        pltpu.sync_copy(obufs[slot], out_slice(g))

        @pl.when(g + 2 < NCHUNK)
        def _():
            issue(g + 2, slot)

    issue(0, 0)
    issue(1, 1)

    def body(t, carry):
        half(2 * t, 0)
        half(2 * t + 1, 1)
        return carry

    lax.fori_loop(0, NCHUNK // 2, body, 0, unroll=False)


def _sc_embed(rt, ad, ac, sd, ci, fc, ff, dtn, cn, cln, fnn):
    mesh = plsc.VectorSubcoreMesh(
        core_axis_name="c", subcore_axis_name="s",
        num_cores=NC, num_subcores=NS)
    row_bufs = [
        pltpu.VMEM((C, D), jnp.float32),
        pltpu.VMEM((CA, D), jnp.float32),
        pltpu.VMEM((CA, D), jnp.float32),
        pltpu.VMEM((CA, D), jnp.float32),
        pltpu.VMEM((CA, D), jnp.float32),
        pltpu.VMEM((CA, CLASS_D), jnp.float32),
        pltpu.VMEM((CA, FUNC_D), jnp.float32),
    ]
    f = pl.kernel(
        _sc_body,
        out_type=jax.ShapeDtypeStruct((B, D), jnp.float32),
        mesh=mesh,
        scratch_types=[
            pltpu.VMEM((NCHUNK, C), jnp.int32),
            pltpu.VMEM((NCHUNK, CA), jnp.int32),
            pltpu.VMEM((NCHUNK, CA), jnp.int32),
            pltpu.VMEM((NCHUNK, CA), jnp.int32),
            pltpu.VMEM((NCHUNK, CA), jnp.int32),
            pltpu.VMEM((NCHUNK, CA), jnp.int32),
            pltpu.VMEM((NCHUNK, CA), jnp.int32),
            *row_bufs,
            *row_bufs,
            pltpu.VMEM((C, D), jnp.float32),
            pltpu.VMEM((C, D), jnp.float32),
            pltpu.SemaphoreType.DMA,
            pltpu.SemaphoreType.DMA,
            pltpu.SemaphoreType.DMA,
            pltpu.SemaphoreType.DMA,
        ],
        compiler_params=pltpu.CompilerParams(use_tc_tiling_on_sc=False),
    )
    return f(rt, ad, ac, sd, ci, fc, ff, dtn, cn, cln, fnn)


def kernel(rtype_idx, arg_dt_idx, arg_const_idx, stmt_dt_idx, const_idx,
           func_class_idx, func_func_idx, dt_table, const_table,
           class_table, func_table):
    dtn = _renorm_table(dt_table, 128, 512)
    cn = _renorm_table(const_table, 128, 2048)
    cln = _renorm_table(class_table, 128, 2048)
    fnn = _renorm_table(func_table, 384, 2048)

    i32 = jnp.int32
    rt = rtype_idx.astype(i32).reshape(NW, NCHUNK, C)
    ad = arg_dt_idx.astype(i32).reshape(NW, NCHUNK, CA)
    ac = arg_const_idx.astype(i32).reshape(NW, NCHUNK, CA)
    sd = stmt_dt_idx.astype(i32).reshape(NW, NCHUNK, CA)
    ci = const_idx.astype(i32).reshape(NW, NCHUNK, CA)
    fc = func_class_idx.astype(i32).reshape(NW, NCHUNK, CA)
    ff = func_func_idx.astype(i32).reshape(NW, NCHUNK, CA)

    return _sc_embed(rt, ad, ac, sd, ci, fc, ff, dtn, cn, cln, fnn)
